# baseline (speedup 1.0000x reference)
import functools

import jax
import jax.numpy as jnp
from jax import lax
from jax.experimental import pallas as pl
from jax.experimental.pallas import tpu as pltpu
from jax.experimental.pallas import tpu_sc as plsc

NC = 2
NS = 16
NW = NC * NS
L = 16

ROWS_OUT = 127
N = 32768
NO = N + 1
S = N // L
U = 8
M = S - L
MU = M // U
KROWS = (ROWS_OUT + NW - 1) // NW


def _scan_body(x_hbm, out_hbm, in_a, in_b, out_buf, sem_a, sem_b, sem_o):
    wid = lax.axis_index("s") * NC + lax.axis_index("c")
    lane = lax.iota(jnp.int32, L)
    start_vec = lane * S
    end_vec = start_vec + S
    diag0 = start_vec + lane
    zero_f = jnp.zeros((L,), jnp.float32)

    ins = [in_a, in_b]
    sems = [sem_a, sem_b]

    def pass1(in_row):
        def p1(i, accs):
            g0 = i * U
            return tuple(
                accs[u] + plsc.load_gather(in_row, [diag0 + (g0 + u)])
                for u in range(U)
            )

        accs = lax.fori_loop(0, MU, p1, tuple(zero_f for _ in range(U)))
        seg = accs[0]
        for u in range(1, U):
            seg = seg + accs[u]

        presum = zero_f
        for i in range(M, S):
            w = (diag0 + i) >= end_vec
            addr = jnp.where(w, diag0 + (i - S), diag0 + i)
            v = plsc.load_gather(in_row, [addr])
            seg = seg + v
            presum = presum + jnp.where(w, v, zero_f)

        off = plsc.cumsum(seg) - seg
        return off, presum

    def pass2(in_row, off, presum):
        diag1 = diag0 + 1

        def p2(i, run):
            g0 = i * U
            vs = [plsc.load_gather(in_row, [diag0 + (g0 + u)]) for u in range(U)]
            pref = [vs[0]]
            for u in range(1, U):
                pref.append(pref[u - 1] + vs[u])
            plsc.store_scatter(out_buf, [diag1 + g0], run + vs[0])
            for u in range(1, U):
                plsc.store_scatter(out_buf, [diag1 + (g0 + u)], run + pref[u])
            return run + pref[U - 1]

        run = lax.fori_loop(0, MU, p2, off + presum)

        for i in range(M, S):
            w = (diag0 + i) >= end_vec
            wfirst = (diag0 + i) == end_vec
            addr = jnp.where(w, diag0 + (i - S), diag0 + i)
            run = jnp.where(wfirst, off, run)
            v = plsc.load_gather(in_row, [addr])
            run = run + v
            plsc.store_scatter(out_buf, [addr + 1], run)

        plsc.store_scatter(
            out_buf, [jnp.broadcast_to(0, (L,))], zero_f, mask=lane == 0
        )

    d_in = pltpu.async_copy(x_hbm.at[wid], in_a, sem_a)
    prev_out = None
    for k in range(KROWS):
        row = wid + NW * k
        d_in.wait()
        if k + 1 < KROWS:
            nxt = (k + 1) % 2
            d_in = pltpu.async_copy(
                x_hbm.at[wid + NW * (k + 1)], ins[nxt], sems[nxt]
            )
        cur = ins[k % 2]

        if k < KROWS - 1:
            off, presum = pass1(cur)
            if prev_out is not None:
                prev_out.wait()
            pass2(cur, off, presum)
            prev_out = pltpu.async_copy(out_buf, out_hbm.at[row], sem_o)
        else:
            prev = prev_out

            @pl.when(row < ROWS_OUT)
            def _():
                off, presum = pass1(cur)
                prev.wait()
                pass2(cur, off, presum)
                pltpu.async_copy(out_buf, out_hbm.at[row], sem_o).wait()

            @pl.when(row >= ROWS_OUT)
            def _():
                prev.wait()


@jax.jit
def _exclusive_scan(x):
    mesh = plsc.VectorSubcoreMesh(core_axis_name="c", subcore_axis_name="s")
    return pl.kernel(
        _scan_body,
        out_type=jax.ShapeDtypeStruct((ROWS_OUT, NO), jnp.float32),
        mesh=mesh,
        scratch_types=[
            pltpu.VMEM((N,), jnp.float32),
            pltpu.VMEM((N,), jnp.float32),
            pltpu.VMEM((NO,), jnp.float32),
            pltpu.SemaphoreType.DMA,
            pltpu.SemaphoreType.DMA,
            pltpu.SemaphoreType.DMA,
        ],
        compiler_params=pltpu.CompilerParams(needs_layout_passes=False),
    )(x)


def kernel(x):
    return _exclusive_scan(x)


# speedup vs baseline: 1.0320x; 1.0320x over previous
import functools

import jax
import jax.numpy as jnp
from jax import lax
from jax.experimental import pallas as pl
from jax.experimental.pallas import tpu as pltpu
from jax.experimental.pallas import tpu_sc as plsc

NC = 2
NS = 16
NW = NC * NS
L = 16

ROWS_OUT = 127
N = 32768
NO = N + 1
S = N // L
U = 16
M = S - L
MU = M // U
KROWS = (ROWS_OUT + NW - 1) // NW


def _scan_body(x_hbm, out_hbm, in_a, in_b, out_buf, sem_a, sem_b, sem_o):
    wid = lax.axis_index("s") * NC + lax.axis_index("c")
    lane = lax.iota(jnp.int32, L)
    start_vec = lane * S
    end_vec = start_vec + S
    diag0 = start_vec + lane
    zero_f = jnp.zeros((L,), jnp.float32)

    ins = [in_a, in_b]
    sems = [sem_a, sem_b]

    def pass1(in_row):
        def p1(i, accs):
            g0 = i * U
            return tuple(
                accs[u] + plsc.load_gather(in_row, [diag0 + (g0 + u)])
                for u in range(U)
            )

        accs = lax.fori_loop(0, MU, p1, tuple(zero_f for _ in range(U)))
        seg = accs[0]
        for u in range(1, U):
            seg = seg + accs[u]

        presum = zero_f
        for i in range(M, S):
            w = (diag0 + i) >= end_vec
            addr = jnp.where(w, diag0 + (i - S), diag0 + i)
            v = plsc.load_gather(in_row, [addr])
            seg = seg + v
            presum = presum + jnp.where(w, v, zero_f)

        off = plsc.cumsum(seg) - seg
        return off, presum

    def pass2(in_row, off, presum):
        diag1 = diag0 + 1

        def p2(i, run):
            g0 = i * U
            vs = [plsc.load_gather(in_row, [diag0 + (g0 + u)]) for u in range(U)]
            pref = [vs[0]]
            for u in range(1, U):
                pref.append(pref[u - 1] + vs[u])
            plsc.store_scatter(out_buf, [diag1 + g0], run + vs[0])
            for u in range(1, U):
                plsc.store_scatter(out_buf, [diag1 + (g0 + u)], run + pref[u])
            return run + pref[U - 1]

        run = lax.fori_loop(0, MU, p2, off + presum)

        for i in range(M, S):
            w = (diag0 + i) >= end_vec
            wfirst = (diag0 + i) == end_vec
            addr = jnp.where(w, diag0 + (i - S), diag0 + i)
            run = jnp.where(wfirst, off, run)
            v = plsc.load_gather(in_row, [addr])
            run = run + v
            plsc.store_scatter(out_buf, [addr + 1], run)

        plsc.store_scatter(
            out_buf, [jnp.broadcast_to(0, (L,))], zero_f, mask=lane == 0
        )

    d_in = pltpu.async_copy(x_hbm.at[wid], in_a, sem_a)
    prev_out = None
    for k in range(KROWS):
        row = wid + NW * k
        d_in.wait()
        if k + 1 < KROWS:
            nxt = (k + 1) % 2
            d_in = pltpu.async_copy(
                x_hbm.at[wid + NW * (k + 1)], ins[nxt], sems[nxt]
            )
        cur = ins[k % 2]

        if k < KROWS - 1:
            off, presum = pass1(cur)
            if prev_out is not None:
                prev_out.wait()
            pass2(cur, off, presum)
            prev_out = pltpu.async_copy(out_buf, out_hbm.at[row], sem_o)
        else:
            prev = prev_out

            @pl.when(row < ROWS_OUT)
            def _():
                off, presum = pass1(cur)
                prev.wait()
                pass2(cur, off, presum)
                pltpu.async_copy(out_buf, out_hbm.at[row], sem_o).wait()

            @pl.when(row >= ROWS_OUT)
            def _():
                prev.wait()


@jax.jit
def _exclusive_scan(x):
    mesh = plsc.VectorSubcoreMesh(core_axis_name="c", subcore_axis_name="s")
    return pl.kernel(
        _scan_body,
        out_type=jax.ShapeDtypeStruct((ROWS_OUT, NO), jnp.float32),
        mesh=mesh,
        scratch_types=[
            pltpu.VMEM((N,), jnp.float32),
            pltpu.VMEM((N,), jnp.float32),
            pltpu.VMEM((NO,), jnp.float32),
            pltpu.SemaphoreType.DMA,
            pltpu.SemaphoreType.DMA,
            pltpu.SemaphoreType.DMA,
        ],
        compiler_params=pltpu.CompilerParams(needs_layout_passes=False),
    )(x)


def kernel(x):
    return _exclusive_scan(x)
